# baseline (speedup 1.0000x reference)
import functools

import jax
import jax.numpy as jnp
import numpy as np
from jax import lax
from jax.experimental import pallas as pl
from jax.experimental.pallas import tpu as pltpu
from jax.experimental.pallas import tpu_sc as plsc

_NUM_PER_CLS = np.array([5000,4773,4556,4349,4151,3963,3782,3611,3447,3290,3141,2998,2862,2732,2608,2489,2376,2268,2165,2067,1973,1883,1798,1716,1638,1564,1493,1425,1360,1298,1239,1183,1129,1078,1029,982,937,895,854,815,778,743,709,677,646,617,589,562,536,512,489,466,445,425,406,387,370,353,337,321,307,293,280,267,255,243,232,222,212,202,193,184,176,168,160,153,146,139,133,127,121,116,110,105,101,96,92,88,84,80,76,73,70,66,63,60,58,55,52,50], dtype=np.float64)
_MAX_M = 0.5
_S = 30.0
_m = 1.0 / np.sqrt(np.sqrt(_NUM_PER_CLS))
_m = _m * (_MAX_M / np.max(_m))
_beta = 0.9999
_eff = 1.0 - np.power(_beta, _NUM_PER_CLS)
_w = (1.0 - _beta) / np.array(_eff)
_w = _w / np.sum(_w) * len(_NUM_PER_CLS)
_SMT_TBL = jnp.asarray(np.pad(_S * _m, (0, 28)), dtype=jnp.float32)
_W_TBL = jnp.asarray(np.pad(_w, (0, 28)), dtype=jnp.float32)


def _gather_body(nchunk, t_hbm, m_hbm, w_hbm, mt_out, wt_out,
                 t_v, mv_v, wv_v, sem):
    wid = lax.axis_index("s") * 2 + lax.axis_index("c")
    base = wid * nchunk
    pltpu.sync_copy(t_hbm.at[pl.ds(base, nchunk)], t_v)
    cps = []
    for j in range(nchunk):
        cps.append(pltpu.async_copy(m_hbm.at[t_v.at[j]], mv_v.at[j], sem))
        cps.append(pltpu.async_copy(w_hbm.at[t_v.at[j]], wv_v.at[j], sem))
    for cp in cps:
        cp.wait()
    pltpu.sync_copy(mv_v, mt_out.at[pl.ds(base, nchunk)])
    pltpu.sync_copy(wv_v, wt_out.at[pl.ds(base, nchunk)])


def _sc_gather(target):
    b = target.shape[0]
    nchunk = b // (32 * 128)
    f32 = jnp.float32
    mesh = plsc.VectorSubcoreMesh(core_axis_name="c", subcore_axis_name="s")
    mt, wt = pl.kernel(
        functools.partial(_gather_body, nchunk),
        mesh=mesh,
        out_type=[jax.ShapeDtypeStruct((b // 128, 128), f32)] * 2,
        scratch_types=[
            pltpu.VMEM((nchunk, 128), jnp.int32),
            pltpu.VMEM((nchunk, 128), f32),
            pltpu.VMEM((nchunk, 128), f32),
            pltpu.SemaphoreType.DMA,
        ],
    )(target.reshape(b // 128, 128), _SMT_TBL, _W_TBL)
    return mt.reshape(b), wt.reshape(b)


def _dense_body(x_ref, t_ref, rmx_ref, sum0_ref, a_ref):
    x = x_ref[...]
    t = t_ref[...]
    bm, c = x.shape
    rowmax = jnp.max(x, axis=1, keepdims=True)
    e = jnp.exp(_S * (x - rowmax))
    sum0_ref[...] = jnp.sum(e, axis=1)
    rmx_ref[...] = rowmax[:, 0]
    j = lax.broadcasted_iota(jnp.int32, (bm, c), 1)
    a_ref[...] = jnp.sum(jnp.where(j == t, x, 0.0), axis=1)


def _combine_body(rmx_ref, sum0_ref, a_ref, smt_ref, wt_ref, out_ref):
    rowmax = _S * rmx_ref[...]
    sum0 = sum0_ref[...]
    a = _S * a_ref[...]
    smt = smt_ref[...]
    wt = wt_ref[...]
    et = jnp.exp(a - rowmax)
    sum_corr = jnp.maximum(sum0 - et, 0.0) + et * jnp.exp(-smt)
    ce = rowmax + jnp.log(sum_corr) - a + smt
    out_ref[0, 0] = jnp.sum(wt * ce) / jnp.sum(wt)


@jax.jit
def kernel(x, target):
    b, c = x.shape
    bm = 2048
    smt, wt = _sc_gather(target)
    rmx, sum0, a_raw = pl.pallas_call(
        _dense_body,
        grid=(b // bm,),
        in_specs=[
            pl.BlockSpec((bm, c), lambda i: (i, 0)),
            pl.BlockSpec((bm, 1), lambda i: (i, 0)),
        ],
        out_specs=[pl.BlockSpec((bm,), lambda i: (i,))] * 3,
        out_shape=[jax.ShapeDtypeStruct((b,), jnp.float32)] * 3,
        compiler_params=pltpu.CompilerParams(
            dimension_semantics=("arbitrary",),
        ),
    )(x, target.reshape(b, 1))
    out = pl.pallas_call(
        _combine_body,
        out_specs=pl.BlockSpec(memory_space=pltpu.SMEM),
        out_shape=jax.ShapeDtypeStruct((1, 1), jnp.float32),
    )(rmx, sum0, a_raw, smt, wt)
    return out[0, 0]


# speedup vs baseline: 1.8123x; 1.8123x over previous
import functools

import jax
import jax.numpy as jnp
import numpy as np
from jax import lax
from jax.experimental import pallas as pl
from jax.experimental.pallas import tpu as pltpu
from jax.experimental.pallas import tpu_sc as plsc

_NUM_PER_CLS = np.array([5000,4773,4556,4349,4151,3963,3782,3611,3447,3290,3141,2998,2862,2732,2608,2489,2376,2268,2165,2067,1973,1883,1798,1716,1638,1564,1493,1425,1360,1298,1239,1183,1129,1078,1029,982,937,895,854,815,778,743,709,677,646,617,589,562,536,512,489,466,445,425,406,387,370,353,337,321,307,293,280,267,255,243,232,222,212,202,193,184,176,168,160,153,146,139,133,127,121,116,110,105,101,96,92,88,84,80,76,73,70,66,63,60,58,55,52,50], dtype=np.float64)
_MAX_M = 0.5
_S = 30.0
_m = 1.0 / np.sqrt(np.sqrt(_NUM_PER_CLS))
_m = _m * (_MAX_M / np.max(_m))
_beta = 0.9999
_eff = 1.0 - np.power(_beta, _NUM_PER_CLS)
_w = (1.0 - _beta) / np.array(_eff)
_w = _w / np.sum(_w) * len(_NUM_PER_CLS)
_SMT_TBL = jnp.asarray(np.tile(np.pad(_S * _m, (0, 28)), 32), dtype=jnp.float32)
_W_TBL = jnp.asarray(np.tile(np.pad(_w, (0, 28)), 32), dtype=jnp.float32)


def _gather_body(nchunk, t_hbm, m_hbm, w_hbm, mt_out, wt_out,
                 t_v, ti_v, mv_v, wv_v, sem):
    wid = lax.axis_index("s") * 2 + lax.axis_index("c")
    base = wid * nchunk
    pltpu.sync_copy(t_hbm.at[pl.ds(base, nchunk)], t_v)
    off = wid * 128
    for j in range(nchunk):
        for k in range(8):
            ti_v.at[j][pl.ds(k * 16, 16)] = t_v.at[j][pl.ds(k * 16, 16)] + off
    cps = []
    for j in range(nchunk):
        cps.append(pltpu.async_copy(m_hbm.at[ti_v.at[j]], mv_v.at[j], sem))
        cps.append(pltpu.async_copy(w_hbm.at[ti_v.at[j]], wv_v.at[j], sem))
    for cp in cps:
        cp.wait()
    pltpu.sync_copy(mv_v, mt_out.at[pl.ds(base, nchunk)])
    pltpu.sync_copy(wv_v, wt_out.at[pl.ds(base, nchunk)])


def _sc_gather(target):
    b = target.shape[0]
    nchunk = b // (32 * 128)
    f32 = jnp.float32
    mesh = plsc.VectorSubcoreMesh(core_axis_name="c", subcore_axis_name="s")
    mt, wt = pl.kernel(
        functools.partial(_gather_body, nchunk),
        mesh=mesh,
        out_type=[jax.ShapeDtypeStruct((b // 128, 128), f32)] * 2,
        scratch_types=[
            pltpu.VMEM((nchunk, 128), jnp.int32),
            pltpu.VMEM((nchunk, 128), jnp.int32),
            pltpu.VMEM((nchunk, 128), f32),
            pltpu.VMEM((nchunk, 128), f32),
            pltpu.SemaphoreType.DMA,
        ],
    )(target.reshape(b // 128, 128), _SMT_TBL, _W_TBL)
    return mt.reshape(b), wt.reshape(b)


def _dense_body(x_ref, t_ref, rmx_ref, sum0_ref, a_ref):
    x = x_ref[...]
    t = t_ref[...]
    bm, c = x.shape
    rowmax = jnp.max(x, axis=1, keepdims=True)
    e = jnp.exp(_S * (x - rowmax))
    sum0_ref[...] = jnp.sum(e, axis=1)
    rmx_ref[...] = rowmax[:, 0]
    j = lax.broadcasted_iota(jnp.int32, (bm, c), 1)
    a_ref[...] = jnp.sum(jnp.where(j == t, x, 0.0), axis=1)


def _combine_body(rmx_ref, sum0_ref, a_ref, smt_ref, wt_ref, out_ref):
    rowmax = _S * rmx_ref[...]
    sum0 = sum0_ref[...]
    a = _S * a_ref[...]
    smt = smt_ref[...]
    wt = wt_ref[...]
    et = jnp.exp(a - rowmax)
    sum_corr = jnp.maximum(sum0 - et, 0.0) + et * jnp.exp(-smt)
    ce = rowmax + jnp.log(sum_corr) - a + smt
    out_ref[0, 0] = jnp.sum(wt * ce) / jnp.sum(wt)


@jax.jit
def kernel(x, target):
    b, c = x.shape
    bm = 2048
    smt, wt = _sc_gather(target)
    rmx, sum0, a_raw = pl.pallas_call(
        _dense_body,
        grid=(b // bm,),
        in_specs=[
            pl.BlockSpec((bm, c), lambda i: (i, 0)),
            pl.BlockSpec((bm, 1), lambda i: (i, 0)),
        ],
        out_specs=[pl.BlockSpec((bm,), lambda i: (i,))] * 3,
        out_shape=[jax.ShapeDtypeStruct((b,), jnp.float32)] * 3,
        compiler_params=pltpu.CompilerParams(
            dimension_semantics=("arbitrary",),
        ),
    )(x, target.reshape(b, 1))
    out = pl.pallas_call(
        _combine_body,
        out_specs=pl.BlockSpec(memory_space=pltpu.SMEM),
        out_shape=jax.ShapeDtypeStruct((1, 1), jnp.float32),
    )(rmx, sum0, a_raw, smt, wt)
    return out[0, 0]


# speedup vs baseline: 3.4378x; 1.8970x over previous
import functools

import jax
import jax.numpy as jnp
import numpy as np
from jax import lax
from jax.experimental import pallas as pl
from jax.experimental.pallas import tpu as pltpu

_NUM_PER_CLS = np.array([5000,4773,4556,4349,4151,3963,3782,3611,3447,3290,3141,2998,2862,2732,2608,2489,2376,2268,2165,2067,1973,1883,1798,1716,1638,1564,1493,1425,1360,1298,1239,1183,1129,1078,1029,982,937,895,854,815,778,743,709,677,646,617,589,562,536,512,489,466,445,425,406,387,370,353,337,321,307,293,280,267,255,243,232,222,212,202,193,184,176,168,160,153,146,139,133,127,121,116,110,105,101,96,92,88,84,80,76,73,70,66,63,60,58,55,52,50], dtype=np.float64)
_MAX_M = 0.5
_S = 30.0
_m = 1.0 / np.sqrt(np.sqrt(_NUM_PER_CLS))
_m = _m * (_MAX_M / np.max(_m))
_beta = 0.9999
_eff = 1.0 - np.power(_beta, _NUM_PER_CLS)
_w = (1.0 - _beta) / np.array(_eff)
_w = _w / np.sum(_w) * len(_NUM_PER_CLS)

def _pack_tables():
    smt16 = (np.float32(_S * _m).view(np.uint32) >> 16).astype(np.uint32)
    w16 = (np.float32(_w).view(np.uint32) >> 16).astype(np.uint32)
    packed = ((smt16 << 16) | w16).astype(np.uint32)
    return jnp.asarray(packed.view(np.float32)[None, :])

_MW_PACKED = _pack_tables()


def _ldam_body(nsteps, x_ref, t_ref, mw_ref, out_ref, acc_ref):
    i = pl.program_id(0)
    x = x_ref[...]
    t = t_ref[...]
    bm, c = x.shape
    j = lax.broadcasted_iota(jnp.int32, (bm, c), 1)
    onehot = j == t

    rowmax = jnp.max(x, axis=1)
    e = jnp.exp(_S * (x - rowmax[:, None]))
    sum0 = jnp.sum(e, axis=1)
    xt = jnp.sum(jnp.where(onehot, x, 0.0), axis=1)
    mw = jnp.sum(jnp.where(onehot, mw_ref[...], 0.0), axis=1)

    bits = lax.bitcast_convert_type(mw, jnp.uint32)
    smt = lax.bitcast_convert_type(bits & jnp.uint32(0xFFFF0000), jnp.float32)
    wt = lax.bitcast_convert_type(bits << 16, jnp.float32)

    a = _S * xt
    rm = _S * rowmax
    et = jnp.exp(a - rm)
    sum_corr = jnp.maximum(sum0 - et, 0.0) + et * jnp.exp(-smt)
    ce = rm + jnp.log(sum_corr) - a + smt
    num = jnp.sum(wt * ce)
    den = jnp.sum(wt)

    @pl.when(i == 0)
    def _():
        acc_ref[0] = num
        acc_ref[1] = den

    @pl.when(i > 0)
    def _():
        acc_ref[0] += num
        acc_ref[1] += den

    @pl.when(i == nsteps - 1)
    def _():
        out_ref[0, 0] = acc_ref[0] / acc_ref[1]


@jax.jit
def kernel(x, target):
    b, c = x.shape
    bm = 2048
    nsteps = b // bm
    out = pl.pallas_call(
        functools.partial(_ldam_body, nsteps),
        grid=(nsteps,),
        in_specs=[
            pl.BlockSpec((bm, c), lambda i: (i, 0)),
            pl.BlockSpec((bm, 1), lambda i: (i, 0)),
            pl.BlockSpec((1, c), lambda i: (0, 0)),
        ],
        out_specs=pl.BlockSpec(memory_space=pltpu.SMEM),
        out_shape=jax.ShapeDtypeStruct((1, 1), jnp.float32),
        scratch_shapes=[pltpu.SMEM((2,), jnp.float32)],
        compiler_params=pltpu.CompilerParams(
            dimension_semantics=("arbitrary",),
        ),
    )(x, target.reshape(b, 1), _MW_PACKED)
    return out[0, 0]


# speedup vs baseline: 3.9371x; 1.1452x over previous
import functools

import jax
import jax.numpy as jnp
import numpy as np
from jax import lax
from jax.experimental import pallas as pl
from jax.experimental.pallas import tpu as pltpu

_NUM_PER_CLS = np.array([5000,4773,4556,4349,4151,3963,3782,3611,3447,3290,3141,2998,2862,2732,2608,2489,2376,2268,2165,2067,1973,1883,1798,1716,1638,1564,1493,1425,1360,1298,1239,1183,1129,1078,1029,982,937,895,854,815,778,743,709,677,646,617,589,562,536,512,489,466,445,425,406,387,370,353,337,321,307,293,280,267,255,243,232,222,212,202,193,184,176,168,160,153,146,139,133,127,121,116,110,105,101,96,92,88,84,80,76,73,70,66,63,60,58,55,52,50], dtype=np.float64)
_MAX_M = 0.5
_S = 30.0
_m = 1.0 / np.sqrt(np.sqrt(_NUM_PER_CLS))
_m = _m * (_MAX_M / np.max(_m))
_M_LIST = jnp.asarray(_m[None, :], dtype=jnp.float32)
_beta = 0.9999
_eff = 1.0 - np.power(_beta, _NUM_PER_CLS)
_w = (1.0 - _beta) / np.array(_eff)
_w = _w / np.sum(_w) * len(_NUM_PER_CLS)
_W_CLS = jnp.asarray(_w[None, :], dtype=jnp.float32)


def _ldam_body(nsteps, x_ref, t_ref, m_ref, w_ref, out_ref, acc_ref):
    i = pl.program_id(0)
    x = x_ref[...]
    t = t_ref[...]
    bm, c = x.shape
    j = lax.broadcasted_iota(jnp.int32, (bm, c), 1)
    onehot = j == t
    m = m_ref[...]
    logits = _S * jnp.where(onehot, x - m, x)
    rowmax = jnp.max(logits, axis=1, keepdims=True)
    e = jnp.exp(logits - rowmax)
    sumexp = jnp.sum(e, axis=1)
    lse = rowmax[:, 0] + jnp.log(sumexp)
    tgt_logit = jnp.sum(jnp.where(onehot, logits, 0.0), axis=1)
    ce = lse - tgt_logit
    w = w_ref[...]
    wt = jnp.sum(jnp.where(onehot, w, jnp.zeros_like(w)), axis=1)
    num = jnp.sum(wt * ce)
    den = jnp.sum(wt)

    @pl.when(i == 0)
    def _():
        acc_ref[0] = num
        acc_ref[1] = den

    @pl.when(i > 0)
    def _():
        acc_ref[0] += num
        acc_ref[1] += den

    @pl.when(i == nsteps - 1)
    def _():
        out_ref[0, 0] = acc_ref[0] / acc_ref[1]


@jax.jit
def kernel(x, target):
    b, c = x.shape
    bm = 4096
    nsteps = b // bm
    t2 = target.reshape(b, 1)
    out = pl.pallas_call(
        functools.partial(_ldam_body, nsteps),
        grid=(nsteps,),
        in_specs=[
            pl.BlockSpec((bm, c), lambda i: (i, 0)),
            pl.BlockSpec((bm, 1), lambda i: (i, 0)),
            pl.BlockSpec((1, c), lambda i: (0, 0)),
            pl.BlockSpec((1, c), lambda i: (0, 0)),
        ],
        out_specs=pl.BlockSpec(memory_space=pltpu.SMEM),
        out_shape=jax.ShapeDtypeStruct((1, 1), jnp.float32),
        scratch_shapes=[pltpu.SMEM((2,), jnp.float32)],
        compiler_params=pltpu.CompilerParams(
            dimension_semantics=("arbitrary",),
        ),
    )(x, t2, _M_LIST, _W_CLS)
    return out[0, 0]
